# Initial kernel scaffold; baseline (speedup 1.0000x reference)
import functools

import jax
import jax.numpy as jnp
from jax import lax
from jax.experimental import pallas as pl
from jax.experimental.pallas import tpu as pltpu
from jax.experimental.pallas import tpu_sc as plsc

NC = 2
NS = 16
NW = NC * NS
CH = 128
LANES = 16


def _ceil_to(x, m):
    return (x + m - 1) // m * m


@functools.lru_cache(maxsize=None)
def _make_edge_seg_sum(n_nodes, feat, cpt, rows_pad, gather):
    mesh = plsc.VectorSubcoreMesh(
        core_axis_name="c", subcore_axis_name="s", num_cores=NC, num_subcores=NS
    )
    ro_per_tile = n_nodes // NS
    ro_sizes = []
    r = ro_per_tile
    while r > 0:
        ro_sizes.append(min(r, 128))
        r -= min(r, 128)
    z_per_tile = rows_pad // NS
    assert z_per_tile % 128 == 0

    scratch = [
        pltpu.VMEM((cpt, CH), jnp.int32),
        pltpu.VMEM((CH, feat), jnp.float32),
        pltpu.VMEM((128, feat), jnp.float32),
        pltpu.VMEM_SHARED((rows_pad, feat), jnp.float32),
    ]
    if gather:
        scratch.insert(0, pltpu.VMEM((cpt, CH), jnp.int32))

    def body(*refs):
        if gather:
            (src_hbm, dst_hbm, tab_hbm, out_hbm,
             src_v, dst_v, rows_v, tmp_v, acc) = refs
        else:
            (dst_hbm, tab_hbm, out_hbm, dst_v, rows_v, tmp_v, acc) = refs
        c = lax.axis_index("c")
        s = lax.axis_index("s")
        w = c * NS + s

        def zrow(i, carry):
            for g in range(feat // LANES):
                tmp_v[i, pl.ds(g * LANES, LANES)] = jnp.zeros(
                    (LANES,), jnp.float32)
            return carry
        lax.fori_loop(0, 128, zrow, 0)
        for k in range(z_per_tile // 128):
            pltpu.sync_copy(tmp_v, acc.at[pl.ds(s * z_per_tile + k * 128, 128)])

        base = w * cpt
        pltpu.sync_copy(dst_hbm.at[pl.ds(base, cpt)], dst_v)
        if gather:
            pltpu.sync_copy(src_hbm.at[pl.ds(base, cpt)], src_v)
        plsc.subcore_barrier()

        def chunk(j, carry):
            if gather:
                pltpu.sync_copy(tab_hbm.at[src_v.at[j]], rows_v)
            else:
                pltpu.sync_copy(tab_hbm.at[pl.ds((base + j) * CH, CH)], rows_v)
            pltpu.sync_copy(rows_v, acc.at[dst_v.at[j]], add=True)
            return carry
        lax.fori_loop(0, cpt, chunk, 0)
        plsc.subcore_barrier()

        off = 0
        for sz in ro_sizes:
            start = s * ro_per_tile + off
            pltpu.sync_copy(acc.at[pl.ds(start, sz)], tmp_v.at[pl.ds(0, sz)])
            pltpu.sync_copy(tmp_v.at[pl.ds(0, sz)],
                            out_hbm.at[c, pl.ds(start, sz)])
            off += sz

    out_t = jax.ShapeDtypeStruct((NC, n_nodes, feat), jnp.float32)
    return pl.kernel(body, out_type=out_t, mesh=mesh, scratch_types=scratch)


def _layer_body(act, hn_ref, agg_ref, he_ref, ws_ref, wn_ref, we_ref, b_ref,
                out_ref):
    agg = agg_ref[0] + agg_ref[1]
    acc = jnp.dot(hn_ref[...], ws_ref[...], preferred_element_type=jnp.float32)
    acc = acc + jnp.dot(agg, wn_ref[...], preferred_element_type=jnp.float32)
    acc = acc + he_ref[...] * we_ref[...] + b_ref[...]
    if act == "relu":
        acc = jnp.maximum(acc, 0.0)
    elif act == "sigmoid":
        acc = jax.nn.sigmoid(acc)
    out_ref[...] = acc


def _apply_layer(hn, aggP, he128, W, b, act, bm):
    n, d = hn.shape
    dout = W.shape[0]
    WsT = W[:, :d].T
    WnT = W[:, d:2 * d].T
    we = W[:, 2 * d][None, :]
    b2 = b[None, :]
    return pl.pallas_call(
        functools.partial(_layer_body, act),
        grid=(n // bm,),
        in_specs=[
            pl.BlockSpec((bm, d), lambda i: (i, 0)),
            pl.BlockSpec((2, bm, d), lambda i: (0, i, 0)),
            pl.BlockSpec((bm, d), lambda i: (i, 0)),
            pl.BlockSpec((d, dout), lambda i: (0, 0)),
            pl.BlockSpec((d, dout), lambda i: (0, 0)),
            pl.BlockSpec((1, dout), lambda i: (0, 0)),
            pl.BlockSpec((1, dout), lambda i: (0, 0)),
        ],
        out_specs=pl.BlockSpec((bm, dout), lambda i: (i, 0)),
        out_shape=jax.ShapeDtypeStruct((n, dout), jnp.float32),
    )(hn, aggP, he128, WsT, WnT, we, b2)


def kernel(node_feat, edge_index, edge_feat, W1, b1, W2, b2, W3, b3, W4, b4,
           W5, b5, W6, b6, W7, b7):
    n, d = node_feat.shape
    e = edge_index.shape[1]
    cpt = -(-e // (CH * NW))
    n_chunks = cpt * NW
    e_pad = n_chunks * CH
    pad = e_pad - e

    src = edge_index[0]
    dst = edge_index[1]
    srcp = jnp.pad(src, (0, pad)).reshape(n_chunks, CH)
    dstp = jnp.pad(dst, (0, pad), constant_values=n).reshape(n_chunks, CH)
    hep = jnp.pad(edge_feat.astype(jnp.float32), ((0, pad), (0, 15)))
    rows_pad = _ceil_to(n + 1, NS * 128)

    seg_hn = _make_edge_seg_sum(n, d, cpt, rows_pad, True)
    seg_he = _make_edge_seg_sum(n, 16, cpt, rows_pad, False)

    heP = seg_he(dstp, hep)
    he_sum = heP[0, :, 0] + heP[1, :, 0]
    he128 = jnp.broadcast_to(he_sum[:, None], (n, d))

    bm = next(c for c in (512, 400, 256, 200, 128, 80, 40, 16, 8)
              if n % c == 0)

    layers = [(W1, b1, "relu"), (W2, b2, "relu"), (W3, b3, "relu"),
              (W4, b4, "relu"), (W5, b5, "relu"), (W6, b6, "sigmoid"),
              (W7, b7, "none")]
    h = node_feat
    for W, b, act in layers:
        aggP = seg_hn(srcp, dstp, h)
        h = _apply_layer(h, aggP, he128, W, b, act, bm)
    return h


# speedup vs baseline: 2.7018x; 2.7018x over previous
import functools

import jax
import jax.numpy as jnp
from jax import lax
from jax.experimental import pallas as pl
from jax.experimental.pallas import tpu as pltpu
from jax.experimental.pallas import tpu_sc as plsc

NC = 2
NS = 16
NW = NC * NS
CH = 128
LANES = 16


def _ceil_to(x, m):
    return (x + m - 1) // m * m


def _mesh():
    return plsc.VectorSubcoreMesh(
        core_axis_name="c", subcore_axis_name="s", num_cores=NC,
        num_subcores=NS)


def _ro_split(n_nodes):
    ro_per_tile = _ceil_to(-(-n_nodes // NS), 8)
    ro_sizes = []
    r = ro_per_tile
    while r > 0:
        ro_sizes.append(min(r, 128))
        r -= min(r, 128)
    return ro_per_tile, ro_sizes


@functools.lru_cache(maxsize=None)
def _make_edge_seg_sum(n_nodes, feat, cpt, rows_pad):
    ro_per_tile, ro_sizes = _ro_split(n_nodes)
    n_out = NS * ro_per_tile
    z_per_tile = rows_pad // NS
    assert z_per_tile % 128 == 0

    scratch = [
        pltpu.VMEM((cpt, CH), jnp.int32),
        pltpu.VMEM((cpt, CH), jnp.int32),
        pltpu.VMEM((CH, feat), jnp.float32),
        pltpu.VMEM_SHARED((rows_pad, feat), jnp.float32),
    ]

    def body(src_hbm, dst_hbm, tab_hbm, out_hbm, src_v, dst_v, rows_v, acc):
        c = lax.axis_index("c")
        s = lax.axis_index("s")
        w = c * NS + s

        def zrow(i, carry):
            for g in range(feat // LANES):
                rows_v[i, pl.ds(g * LANES, LANES)] = jnp.zeros(
                    (LANES,), jnp.float32)
            return carry
        lax.fori_loop(0, 128, zrow, 0)
        for k in range(z_per_tile // 128):
            pltpu.sync_copy(rows_v,
                            acc.at[pl.ds(s * z_per_tile + k * 128, 128)])

        base = w * cpt
        pltpu.sync_copy(dst_hbm.at[pl.ds(base, cpt)], dst_v)
        pltpu.sync_copy(src_hbm.at[pl.ds(base, cpt)], src_v)
        plsc.subcore_barrier()

        def chunk(j, carry):
            pltpu.sync_copy(tab_hbm.at[src_v.at[j]], rows_v)
            pltpu.sync_copy(rows_v, acc.at[dst_v.at[j]], add=True)
            return carry
        lax.fori_loop(0, cpt, chunk, 0)
        plsc.subcore_barrier()

        off = 0
        for sz in ro_sizes:
            start = s * ro_per_tile + off
            pltpu.sync_copy(acc.at[pl.ds(start, sz)], rows_v.at[pl.ds(0, sz)])
            pltpu.sync_copy(rows_v.at[pl.ds(0, sz)],
                            out_hbm.at[c, pl.ds(start, sz)])
            off += sz

    out_t = jax.ShapeDtypeStruct((NC, n_out, feat), jnp.float32)
    return pl.kernel(body, out_type=out_t, mesh=_mesh(),
                     scratch_types=scratch, name=f"seg_rows_{feat}_{cpt}")


@functools.lru_cache(maxsize=None)
def _make_he_seg_sum(n_nodes, cpt, rows_pad):
    ro_per_tile, _ = _ro_split(n_nodes)
    z_per_tile = rows_pad // NS
    assert z_per_tile % LANES == 0 and ro_per_tile <= z_per_tile

    scratch = [
        pltpu.VMEM((cpt, CH), jnp.float32),
        pltpu.VMEM((cpt, CH), jnp.int32),
        pltpu.VMEM((z_per_tile,), jnp.float32),
        pltpu.VMEM_SHARED((rows_pad,), jnp.float32),
    ]

    def body(val_hbm, dst_hbm, out_hbm, val_v, dst_v, z_v, acc):
        c = lax.axis_index("c")
        s = lax.axis_index("s")
        w = c * NS + s

        def zi(i, carry):
            z_v[pl.ds(i * LANES, LANES)] = jnp.zeros((LANES,), jnp.float32)
            return carry
        lax.fori_loop(0, z_per_tile // LANES, zi, 0)
        pltpu.sync_copy(z_v, acc.at[pl.ds(s * z_per_tile, z_per_tile)])

        base = w * cpt
        pltpu.sync_copy(dst_hbm.at[pl.ds(base, cpt)], dst_v)
        pltpu.sync_copy(val_hbm.at[pl.ds(base, cpt)], val_v)
        plsc.subcore_barrier()

        def chunk(j, carry):
            pltpu.sync_copy(val_v.at[j], acc.at[dst_v.at[j]], add=True)
            return carry
        lax.fori_loop(0, cpt, chunk, 0)
        plsc.subcore_barrier()

        pltpu.sync_copy(acc.at[pl.ds(s * ro_per_tile, ro_per_tile)],
                        z_v.at[pl.ds(0, ro_per_tile)])
        pltpu.sync_copy(z_v.at[pl.ds(0, ro_per_tile)],
                        out_hbm.at[pl.ds((c * NS + s) * ro_per_tile,
                                         ro_per_tile)])

    out_t = jax.ShapeDtypeStruct((NC * NS * ro_per_tile,), jnp.float32)
    return pl.kernel(body, out_type=out_t, mesh=_mesh(),
                     scratch_types=scratch, name=f"seg_elem_{cpt}")


def _layer_body(act, hn_ref, agg_ref, he_ref, ws_ref, wn_ref, we_ref, b_ref,
                out_ref):
    agg = agg_ref[0] + agg_ref[1]
    acc = jnp.dot(hn_ref[...], ws_ref[...], preferred_element_type=jnp.float32,
                  precision=lax.Precision.HIGHEST)
    acc = acc + jnp.dot(agg, wn_ref[...], preferred_element_type=jnp.float32,
                        precision=lax.Precision.HIGHEST)
    acc = acc + he_ref[...] * we_ref[...] + b_ref[...]
    if act == "relu":
        acc = jnp.maximum(acc, 0.0)
    elif act == "sigmoid":
        acc = jax.nn.sigmoid(acc)
    out_ref[...] = acc


def _apply_layer(hn, aggP, he128, W, b, act, bm):
    n, d = hn.shape
    dout = W.shape[0]
    WsT = W[:, :d].T
    WnT = W[:, d:2 * d].T
    we = W[:, 2 * d][None, :]
    b2 = b[None, :]
    return pl.pallas_call(
        functools.partial(_layer_body, act),
        grid=(n // bm,),
        in_specs=[
            pl.BlockSpec((bm, d), lambda i: (i, 0)),
            pl.BlockSpec((2, bm, d), lambda i: (0, i, 0)),
            pl.BlockSpec((bm, d), lambda i: (i, 0)),
            pl.BlockSpec((d, dout), lambda i: (0, 0)),
            pl.BlockSpec((d, dout), lambda i: (0, 0)),
            pl.BlockSpec((1, dout), lambda i: (0, 0)),
            pl.BlockSpec((1, dout), lambda i: (0, 0)),
        ],
        out_specs=pl.BlockSpec((bm, dout), lambda i: (i, 0)),
        out_shape=jax.ShapeDtypeStruct((n, dout), jnp.float32),
    )(hn, aggP, he128, WsT, WnT, we, b2)


def kernel(node_feat, edge_index, edge_feat, W1, b1, W2, b2, W3, b3, W4, b4,
           W5, b5, W6, b6, W7, b7):
    n, d = node_feat.shape
    e = edge_index.shape[1]
    cpt = _ceil_to(-(-e // (CH * NW)), 8)
    n_chunks = cpt * NW
    e_pad = n_chunks * CH
    pad = e_pad - e

    src = edge_index[0]
    dst = edge_index[1]
    srcp = jnp.pad(src, (0, pad)).reshape(n_chunks, CH)
    dstp = jnp.pad(dst, (0, pad), constant_values=n).reshape(n_chunks, CH)
    hev2d = jnp.pad(edge_feat[:, 0].astype(jnp.float32),
                    (0, pad)).reshape(n_chunks, CH)
    ro_per_tile, _ = _ro_split(n)
    rows_pad = _ceil_to(max(n + 1, NS * ro_per_tile), NS * 128)

    seg_hn = _make_edge_seg_sum(n, d, cpt, rows_pad)
    seg_he = _make_he_seg_sum(n, cpt, rows_pad)

    he_flat = seg_he(hev2d, dstp)
    heP = he_flat.reshape(NC, NS * ro_per_tile)
    he_sum = heP[0, :n] + heP[1, :n]
    he128 = jnp.broadcast_to(he_sum[:, None], (n, d))

    bm = next(c for c in (512, 400, 256, 200, 128, 80, 40, 16, 8)
              if n % c == 0)

    layers = [(W1, b1, "relu"), (W2, b2, "relu"), (W3, b3, "relu"),
              (W4, b4, "relu"), (W5, b5, "relu"), (W6, b6, "sigmoid"),
              (W7, b7, "none")]
    h = node_feat
    for W, b, act in layers:
        aggP = seg_hn(srcp, dstp, h)
        h = _apply_layer(h, aggP, he128, W, b, act, bm)
    return h


# speedup vs baseline: 2.8344x; 1.0491x over previous
import functools

import jax
import jax.numpy as jnp
from jax import lax
from jax.experimental import pallas as pl
from jax.experimental.pallas import tpu as pltpu
from jax.experimental.pallas import tpu_sc as plsc

NC = 2
NS = 16
NW = NC * NS
CH = 128
LANES = 16


def _ceil_to(x, m):
    return (x + m - 1) // m * m


def _mesh():
    return plsc.VectorSubcoreMesh(
        core_axis_name="c", subcore_axis_name="s", num_cores=NC,
        num_subcores=NS)


def _ro_split(n_nodes):
    ro_per_tile = _ceil_to(-(-n_nodes // NS), 8)
    ro_sizes = []
    r = ro_per_tile
    while r > 0:
        ro_sizes.append(min(r, 128))
        r -= min(r, 128)
    return ro_per_tile, ro_sizes


@functools.lru_cache(maxsize=None)
def _make_edge_seg_sum(n_nodes, feat, cpt, rows_pad):
    ro_per_tile, ro_sizes = _ro_split(n_nodes)
    n_out = NS * ro_per_tile
    z_per_tile = rows_pad // NS
    assert z_per_tile % 128 == 0
    assert cpt % 4 == 0
    cpt2 = cpt // 2
    nhalf = cpt2 // 2

    scratch = [
        pltpu.VMEM((cpt2, CH), jnp.int32),
        pltpu.VMEM((cpt2, CH), jnp.int32),
        pltpu.VMEM((CH, feat), jnp.float32),
        pltpu.VMEM((CH, feat), jnp.float32),
        pltpu.VMEM_SHARED((rows_pad, feat), jnp.float32),
        pltpu.SemaphoreType.DMA,
        pltpu.SemaphoreType.DMA,
        pltpu.SemaphoreType.DMA,
        pltpu.SemaphoreType.DMA,
    ]

    def body(src_hbm, dst_hbm, tab_hbm, out_hbm, src_v, dst_v, rows0, rows1,
             acc, g0, g1, s0, s1):
        c = lax.axis_index("c")
        s = lax.axis_index("s")
        w = c * NS + s

        def zrow(i, carry):
            for g in range(feat // LANES):
                rows0[i, pl.ds(g * LANES, LANES)] = jnp.zeros(
                    (LANES,), jnp.float32)
            return carry
        lax.fori_loop(0, 128, zrow, 0)
        for k in range(z_per_tile // 128):
            pltpu.sync_copy(rows0,
                            acc.at[pl.ds(s * z_per_tile + k * 128, 128)])
        plsc.subcore_barrier()

        def gather(j, buf, sem):
            pltpu.async_copy(tab_hbm.at[src_v.at[j]], buf, sem)

        def scatter(j, buf, sem):
            pltpu.async_copy(buf, acc.at[dst_v.at[j]], sem, add=True)

        def wait_g(buf, sem):
            pltpu.make_async_copy(tab_hbm.at[src_v.at[0]], buf, sem).wait()

        def wait_s(buf, sem):
            pltpu.make_async_copy(buf, acc.at[dst_v.at[0]], sem).wait()

        for p in range(2):
            base = w * cpt + p * cpt2
            pltpu.sync_copy(dst_hbm.at[pl.ds(base, cpt2)], dst_v)
            pltpu.sync_copy(src_hbm.at[pl.ds(base, cpt2)], src_v)
            gather(0, rows0, g0)

            def pipe(j2, carry):
                a = 2 * j2
                wait_g(rows0, g0)

                @pl.when(j2 > 0)
                def _():
                    wait_s(rows1, s1)
                gather(a + 1, rows1, g1)
                scatter(a, rows0, s0)
                wait_g(rows1, g1)

                @pl.when(j2 < nhalf - 1)
                def _():
                    wait_s(rows0, s0)
                    gather(a + 2, rows0, g0)
                scatter(a + 1, rows1, s1)
                return carry
            lax.fori_loop(0, nhalf, pipe, 0)
            wait_s(rows0, s0)
            wait_s(rows1, s1)
        plsc.subcore_barrier()

        off = 0
        for sz in ro_sizes:
            start = s * ro_per_tile + off
            pltpu.sync_copy(acc.at[pl.ds(start, sz)], rows0.at[pl.ds(0, sz)])
            pltpu.sync_copy(rows0.at[pl.ds(0, sz)],
                            out_hbm.at[c, pl.ds(start, sz)])
            off += sz

    out_t = jax.ShapeDtypeStruct((NC, n_out, feat), jnp.float32)
    return pl.kernel(body, out_type=out_t, mesh=_mesh(),
                     scratch_types=scratch, name=f"seg_rows_{feat}_{cpt}")


@functools.lru_cache(maxsize=None)
def _make_he_seg_sum(n_nodes, cpt, rows_pad):
    ro_per_tile, _ = _ro_split(n_nodes)
    z_per_tile = rows_pad // NS
    assert z_per_tile % LANES == 0 and ro_per_tile <= z_per_tile

    scratch = [
        pltpu.VMEM((cpt, CH), jnp.float32),
        pltpu.VMEM((cpt, CH), jnp.int32),
        pltpu.VMEM((z_per_tile,), jnp.float32),
        pltpu.VMEM_SHARED((rows_pad,), jnp.float32),
    ]

    def body(val_hbm, dst_hbm, out_hbm, val_v, dst_v, z_v, acc):
        c = lax.axis_index("c")
        s = lax.axis_index("s")
        w = c * NS + s

        def zi(i, carry):
            z_v[pl.ds(i * LANES, LANES)] = jnp.zeros((LANES,), jnp.float32)
            return carry
        lax.fori_loop(0, z_per_tile // LANES, zi, 0)
        pltpu.sync_copy(z_v, acc.at[pl.ds(s * z_per_tile, z_per_tile)])

        base = w * cpt
        pltpu.sync_copy(dst_hbm.at[pl.ds(base, cpt)], dst_v)
        pltpu.sync_copy(val_hbm.at[pl.ds(base, cpt)], val_v)
        plsc.subcore_barrier()

        def chunk(j, carry):
            pltpu.sync_copy(val_v.at[j], acc.at[dst_v.at[j]], add=True)
            return carry
        lax.fori_loop(0, cpt, chunk, 0)
        plsc.subcore_barrier()

        pltpu.sync_copy(acc.at[pl.ds(s * ro_per_tile, ro_per_tile)],
                        z_v.at[pl.ds(0, ro_per_tile)])
        pltpu.sync_copy(z_v.at[pl.ds(0, ro_per_tile)],
                        out_hbm.at[pl.ds((c * NS + s) * ro_per_tile,
                                         ro_per_tile)])

    out_t = jax.ShapeDtypeStruct((NC * NS * ro_per_tile,), jnp.float32)
    return pl.kernel(body, out_type=out_t, mesh=_mesh(),
                     scratch_types=scratch, name=f"seg_elem_{cpt}")


def _layer_body(act, hn_ref, agg_ref, he_ref, ws_ref, wn_ref, we_ref, b_ref,
                out_ref):
    agg = agg_ref[0] + agg_ref[1]
    acc = jnp.dot(hn_ref[...], ws_ref[...], preferred_element_type=jnp.float32,
                  precision=lax.Precision.HIGHEST)
    acc = acc + jnp.dot(agg, wn_ref[...], preferred_element_type=jnp.float32,
                        precision=lax.Precision.HIGHEST)
    acc = acc + he_ref[...] * we_ref[...] + b_ref[...]
    if act == "relu":
        acc = jnp.maximum(acc, 0.0)
    elif act == "sigmoid":
        acc = jax.nn.sigmoid(acc)
    out_ref[...] = acc


def _apply_layer(hn, aggP, he128, W, b, act, bm):
    n, d = hn.shape
    dout = W.shape[0]
    WsT = W[:, :d].T
    WnT = W[:, d:2 * d].T
    we = W[:, 2 * d][None, :]
    b2 = b[None, :]
    return pl.pallas_call(
        functools.partial(_layer_body, act),
        grid=(n // bm,),
        in_specs=[
            pl.BlockSpec((bm, d), lambda i: (i, 0)),
            pl.BlockSpec((2, bm, d), lambda i: (0, i, 0)),
            pl.BlockSpec((bm, d), lambda i: (i, 0)),
            pl.BlockSpec((d, dout), lambda i: (0, 0)),
            pl.BlockSpec((d, dout), lambda i: (0, 0)),
            pl.BlockSpec((1, dout), lambda i: (0, 0)),
            pl.BlockSpec((1, dout), lambda i: (0, 0)),
        ],
        out_specs=pl.BlockSpec((bm, dout), lambda i: (i, 0)),
        out_shape=jax.ShapeDtypeStruct((n, dout), jnp.float32),
    )(hn, aggP, he128, WsT, WnT, we, b2)


def kernel(node_feat, edge_index, edge_feat, W1, b1, W2, b2, W3, b3, W4, b4,
           W5, b5, W6, b6, W7, b7):
    n, d = node_feat.shape
    e = edge_index.shape[1]
    cpt = _ceil_to(-(-e // (CH * NW)), 8)
    n_chunks = cpt * NW
    e_pad = n_chunks * CH
    pad = e_pad - e

    src = edge_index[0]
    dst = edge_index[1]
    srcp = jnp.pad(src, (0, pad)).reshape(n_chunks, CH)
    dstp = jnp.pad(dst, (0, pad), constant_values=n).reshape(n_chunks, CH)
    hev2d = jnp.pad(edge_feat[:, 0].astype(jnp.float32),
                    (0, pad)).reshape(n_chunks, CH)
    ro_per_tile, _ = _ro_split(n)
    rows_pad = _ceil_to(max(n + 1, NS * ro_per_tile), NS * 128)

    seg_hn = _make_edge_seg_sum(n, d, cpt, rows_pad)
    seg_he = _make_he_seg_sum(n, cpt, rows_pad)

    he_flat = seg_he(hev2d, dstp)
    heP = he_flat.reshape(NC, NS * ro_per_tile)
    he_sum = heP[0, :n] + heP[1, :n]
    he128 = jnp.broadcast_to(he_sum[:, None], (n, d))

    bm = next(c for c in (512, 400, 256, 200, 128, 80, 40, 16, 8)
              if n % c == 0)

    layers = [(W1, b1, "relu"), (W2, b2, "relu"), (W3, b3, "relu"),
              (W4, b4, "relu"), (W5, b5, "relu"), (W6, b6, "sigmoid"),
              (W7, b7, "none")]
    h = node_feat
    for W, b, act in layers:
        aggP = seg_hn(srcp, dstp, h)
        h = _apply_layer(h, aggP, he128, W, b, act, bm)
    return h


# speedup vs baseline: 9.1406x; 3.2249x over previous
import functools

import jax
import jax.numpy as jnp
from jax import lax
from jax.experimental import pallas as pl
from jax.experimental.pallas import tpu as pltpu
from jax.experimental.pallas import tpu_sc as plsc

NC = 2
NS = 16
NW = NC * NS
CH = 128
LANES = 16


def _ceil_to(x, m):
    return (x + m - 1) // m * m


def _mesh():
    return plsc.VectorSubcoreMesh(
        core_axis_name="c", subcore_axis_name="s", num_cores=NC,
        num_subcores=NS)


def _ro_split(n_nodes):
    ro_per_tile = _ceil_to(-(-n_nodes // NS), 8)
    ro_sizes = []
    r = ro_per_tile
    while r > 0:
        ro_sizes.append(min(r, 128))
        r -= min(r, 128)
    return ro_per_tile, ro_sizes


@functools.lru_cache(maxsize=None)
def _make_edge_seg_sum(n_nodes, feat, cpt, rows_pad):
    ro_per_tile, ro_sizes = _ro_split(n_nodes)
    n_out = NS * ro_per_tile
    z_per_tile = rows_pad // NS
    assert z_per_tile % 128 == 0
    assert cpt % 4 == 0
    cpt2 = cpt // 2
    nhalf = cpt2 // 2

    scratch = [
        pltpu.VMEM((cpt2, CH), jnp.int32),
        pltpu.VMEM((cpt2, CH), jnp.int32),
        pltpu.VMEM((CH, feat), jnp.float32),
        pltpu.VMEM((CH, feat), jnp.float32),
        pltpu.VMEM_SHARED((rows_pad, feat), jnp.float32),
        pltpu.SemaphoreType.DMA,
        pltpu.SemaphoreType.DMA,
        pltpu.SemaphoreType.DMA,
        pltpu.SemaphoreType.DMA,
    ]

    def body(src_hbm, dst_hbm, tab_hbm, out_hbm, src_v, dst_v, rows0, rows1,
             acc, g0, g1, s0, s1):
        c = lax.axis_index("c")
        s = lax.axis_index("s")
        w = c * NS + s

        def zrow(i, carry):
            for g in range(feat // LANES):
                rows0[i, pl.ds(g * LANES, LANES)] = jnp.zeros(
                    (LANES,), jnp.float32)
            return carry
        lax.fori_loop(0, 128, zrow, 0)
        for k in range(z_per_tile // 128):
            pltpu.sync_copy(rows0,
                            acc.at[pl.ds(s * z_per_tile + k * 128, 128)])
        plsc.subcore_barrier()

        def gather(j, buf, sem):
            pltpu.async_copy(tab_hbm.at[src_v.at[j]], buf, sem)

        def scatter(j, buf, sem):
            pltpu.async_copy(buf, acc.at[dst_v.at[j]], sem, add=True)

        def wait_g(buf, sem):
            pltpu.make_async_copy(tab_hbm.at[src_v.at[0]], buf, sem).wait()

        def wait_s(buf, sem):
            pltpu.make_async_copy(buf, acc.at[dst_v.at[0]], sem).wait()

        for p in range(2):
            base = w * cpt + p * cpt2
            pltpu.sync_copy(dst_hbm.at[pl.ds(base, cpt2)], dst_v)
            pltpu.sync_copy(src_hbm.at[pl.ds(base, cpt2)], src_v)
            gather(0, rows0, g0)

            def pipe(j2, carry):
                a = 2 * j2
                wait_g(rows0, g0)

                @pl.when(j2 > 0)
                def _():
                    wait_s(rows1, s1)
                gather(a + 1, rows1, g1)
                scatter(a, rows0, s0)
                wait_g(rows1, g1)

                @pl.when(j2 < nhalf - 1)
                def _():
                    wait_s(rows0, s0)
                    gather(a + 2, rows0, g0)
                scatter(a + 1, rows1, s1)
                return carry
            lax.fori_loop(0, nhalf, pipe, 0)
            wait_s(rows0, s0)
            wait_s(rows1, s1)
        plsc.subcore_barrier()

        off = 0
        for sz in ro_sizes:
            start = s * ro_per_tile + off
            pltpu.sync_copy(acc.at[pl.ds(start, sz)], rows0.at[pl.ds(0, sz)])
            pltpu.sync_copy(rows0.at[pl.ds(0, sz)],
                            out_hbm.at[c, pl.ds(start, sz)])
            off += sz

    out_t = jax.ShapeDtypeStruct((NC, n_out, feat), jnp.float32)
    return pl.kernel(body, out_type=out_t, mesh=_mesh(),
                     scratch_types=scratch, name=f"seg_rows_{feat}_{cpt}")


@functools.lru_cache(maxsize=None)
def _make_he_seg_sum(n_nodes, cpt, rows_pad):
    ro_per_tile, _ = _ro_split(n_nodes)
    z_per_tile = rows_pad // NS
    assert z_per_tile % LANES == 0 and ro_per_tile <= z_per_tile

    scratch = [
        pltpu.VMEM((cpt, CH), jnp.float32),
        pltpu.VMEM((cpt, CH), jnp.int32),
        pltpu.VMEM((z_per_tile,), jnp.float32),
        pltpu.VMEM_SHARED((rows_pad,), jnp.float32),
    ]

    def body(val_hbm, dst_hbm, out_hbm, val_v, dst_v, z_v, acc):
        c = lax.axis_index("c")
        s = lax.axis_index("s")
        w = c * NS + s

        def zi(i, carry):
            z_v[pl.ds(i * LANES, LANES)] = jnp.zeros((LANES,), jnp.float32)
            return carry
        lax.fori_loop(0, z_per_tile // LANES, zi, 0)
        pltpu.sync_copy(z_v, acc.at[pl.ds(s * z_per_tile, z_per_tile)])

        base = w * cpt
        pltpu.sync_copy(dst_hbm.at[pl.ds(base, cpt)], dst_v)
        pltpu.sync_copy(val_hbm.at[pl.ds(base, cpt)], val_v)
        plsc.subcore_barrier()

        def chunk(j, carry):
            pltpu.sync_copy(val_v.at[j], acc.at[dst_v.at[j]], add=True)
            return carry
        lax.fori_loop(0, cpt, chunk, 0)
        plsc.subcore_barrier()

        pltpu.sync_copy(acc.at[pl.ds(s * ro_per_tile, ro_per_tile)],
                        z_v.at[pl.ds(0, ro_per_tile)])
        pltpu.sync_copy(z_v.at[pl.ds(0, ro_per_tile)],
                        out_hbm.at[pl.ds((c * NS + s) * ro_per_tile,
                                         ro_per_tile)])

    out_t = jax.ShapeDtypeStruct((NC * NS * ro_per_tile,), jnp.float32)
    return pl.kernel(body, out_type=out_t, mesh=_mesh(),
                     scratch_types=scratch, name=f"seg_elem_{cpt}")


def _layer_body(act, hn_ref, agg_ref, he_ref, ws_ref, wn_ref, we_ref, b_ref,
                out_ref):
    agg = agg_ref[0] + agg_ref[1]
    acc = jnp.dot(hn_ref[...], ws_ref[...], preferred_element_type=jnp.float32,
                  precision=lax.Precision.HIGHEST)
    acc = acc + jnp.dot(agg, wn_ref[...], preferred_element_type=jnp.float32,
                        precision=lax.Precision.HIGHEST)
    acc = acc + he_ref[...] * we_ref[...] + b_ref[...]
    if act == "relu":
        acc = jnp.maximum(acc, 0.0)
    elif act == "sigmoid":
        acc = jax.nn.sigmoid(acc)
    out_ref[...] = acc


def _apply_layer(hn, aggP, he128, W, b, act, bm):
    n, d = hn.shape
    dout = W.shape[0]
    WsT = W[:, :d].T
    WnT = W[:, d:2 * d].T
    we = W[:, 2 * d][None, :]
    b2 = b[None, :]
    return pl.pallas_call(
        functools.partial(_layer_body, act),
        grid=(n // bm,),
        in_specs=[
            pl.BlockSpec((bm, d), lambda i: (i, 0)),
            pl.BlockSpec((2, bm, d), lambda i: (0, i, 0)),
            pl.BlockSpec((bm, d), lambda i: (i, 0)),
            pl.BlockSpec((d, dout), lambda i: (0, 0)),
            pl.BlockSpec((d, dout), lambda i: (0, 0)),
            pl.BlockSpec((1, dout), lambda i: (0, 0)),
            pl.BlockSpec((1, dout), lambda i: (0, 0)),
        ],
        out_specs=pl.BlockSpec((bm, dout), lambda i: (i, 0)),
        out_shape=jax.ShapeDtypeStruct((n, dout), jnp.float32),
    )(hn, aggP, he128, WsT, WnT, we, b2)


def kernel(node_feat, edge_index, edge_feat, W1, b1, W2, b2, W3, b3, W4, b4,
           W5, b5, W6, b6, W7, b7):
    n, d = node_feat.shape
    e = edge_index.shape[1]
    cpt = _ceil_to(-(-e // (CH * NW)), 8)
    n_chunks = cpt * NW
    e_pad = n_chunks * CH
    pad = e_pad - e

    ro_per_tile, _ = _ro_split(n)
    rows_pad = _ceil_to(max(n + 1, NS * ro_per_tile), NS * 128)

    src = edge_index[0]
    dst = edge_index[1]
    pad_idx = jnp.arange(pad, dtype=jnp.int32)
    srcp = jnp.concatenate([src, pad_idx % n]).reshape(n_chunks, CH)
    dstp = jnp.concatenate([dst, n + pad_idx % (rows_pad - n)]
                           ).reshape(n_chunks, CH)
    hev2d = jnp.pad(edge_feat[:, 0].astype(jnp.float32),
                    (0, pad)).reshape(n_chunks, CH)

    seg_hn = _make_edge_seg_sum(n, d, cpt, rows_pad)
    seg_he = _make_he_seg_sum(n, cpt, rows_pad)

    he_flat = seg_he(hev2d, dstp)
    heP = he_flat.reshape(NC, NS * ro_per_tile)
    he_sum = heP[0, :n] + heP[1, :n]
    he128 = jnp.broadcast_to(he_sum[:, None], (n, d))

    bm = next(c for c in (512, 400, 256, 200, 128, 80, 40, 16, 8)
              if n % c == 0)

    layers = [(W1, b1, "relu"), (W2, b2, "relu"), (W3, b3, "relu"),
              (W4, b4, "relu"), (W5, b5, "relu"), (W6, b6, "sigmoid"),
              (W7, b7, "none")]
    h = node_feat
    for W, b, act in layers:
        aggP = seg_hn(srcp, dstp, h)
        h = _apply_layer(h, aggP, he128, W, b, act, bm)
    return h


# speedup vs baseline: 9.3357x; 1.0213x over previous
import functools

import jax
import jax.numpy as jnp
from jax import lax
from jax.experimental import pallas as pl
from jax.experimental.pallas import tpu as pltpu
from jax.experimental.pallas import tpu_sc as plsc

NC = 2
NS = 16
NW = NC * NS
CH = 128
LANES = 16


def _ceil_to(x, m):
    return (x + m - 1) // m * m


def _mesh():
    return plsc.VectorSubcoreMesh(
        core_axis_name="c", subcore_axis_name="s", num_cores=NC,
        num_subcores=NS)


def _ro_split(n_nodes):
    ro_per_tile = _ceil_to(-(-n_nodes // NS), 8)
    ro_sizes = []
    r = ro_per_tile
    while r > 0:
        ro_sizes.append(min(r, 128))
        r -= min(r, 128)
    return ro_per_tile, ro_sizes


@functools.lru_cache(maxsize=None)
def _make_edge_seg_sum(n_nodes, feat, cpt, rows_pad):
    ro_per_tile, ro_sizes = _ro_split(n_nodes)
    n_out = NS * ro_per_tile
    z_per_tile = rows_pad // NS
    assert z_per_tile % 128 == 0
    assert cpt % 4 == 0
    cpt2 = cpt // 2
    nhalf = cpt2 // 2

    scratch = [
        pltpu.VMEM((cpt2, CH), jnp.int32),
        pltpu.VMEM((cpt2, CH), jnp.int32),
        pltpu.VMEM((CH, feat), jnp.float32),
        pltpu.VMEM((CH, feat), jnp.float32),
        pltpu.VMEM_SHARED((rows_pad, feat), jnp.float32),
        pltpu.SemaphoreType.DMA,
        pltpu.SemaphoreType.DMA,
        pltpu.SemaphoreType.DMA,
        pltpu.SemaphoreType.DMA,
    ]

    def body(src_hbm, dst_hbm, tab_hbm, out_hbm, src_v, dst_v, rows0, rows1,
             acc, g0, g1, s0, s1):
        c = lax.axis_index("c")
        s = lax.axis_index("s")
        w = c * NS + s

        def zrow(i, carry):
            for g in range(feat // LANES):
                rows0[i, pl.ds(g * LANES, LANES)] = jnp.zeros(
                    (LANES,), jnp.float32)
            return carry
        lax.fori_loop(0, 128, zrow, 0)
        for k in range(z_per_tile // 128):
            pltpu.sync_copy(rows0,
                            acc.at[pl.ds(s * z_per_tile + k * 128, 128)])
        plsc.subcore_barrier()

        def gather(j, buf, sem):
            pltpu.async_copy(tab_hbm.at[src_v.at[j]], buf, sem)

        def scatter(j, buf, sem):
            pltpu.async_copy(buf, acc.at[dst_v.at[j]], sem, add=True)

        def wait_g(buf, sem):
            pltpu.make_async_copy(tab_hbm.at[src_v.at[0]], buf, sem).wait()

        def wait_s(buf, sem):
            pltpu.make_async_copy(buf, acc.at[dst_v.at[0]], sem).wait()

        for p in range(2):
            base = w * cpt + p * cpt2
            pltpu.sync_copy(dst_hbm.at[pl.ds(base, cpt2)], dst_v)
            pltpu.sync_copy(src_hbm.at[pl.ds(base, cpt2)], src_v)
            gather(0, rows0, g0)

            def pipe(j2, carry):
                a = 2 * j2
                wait_g(rows0, g0)

                @pl.when(j2 > 0)
                def _():
                    wait_s(rows1, s1)
                gather(a + 1, rows1, g1)
                scatter(a, rows0, s0)
                wait_g(rows1, g1)

                @pl.when(j2 < nhalf - 1)
                def _():
                    wait_s(rows0, s0)
                    gather(a + 2, rows0, g0)
                scatter(a + 1, rows1, s1)
                return carry
            lax.fori_loop(0, nhalf, pipe, 0)
            wait_s(rows0, s0)
            wait_s(rows1, s1)
        plsc.subcore_barrier()

        off = 0
        for sz in ro_sizes:
            start = s * ro_per_tile + off
            pltpu.sync_copy(acc.at[pl.ds(start, sz)], rows0.at[pl.ds(0, sz)])
            pltpu.sync_copy(rows0.at[pl.ds(0, sz)],
                            out_hbm.at[c, pl.ds(start, sz)])
            off += sz

    out_t = jax.ShapeDtypeStruct((NC, n_out, feat), jnp.float32)
    return pl.kernel(body, out_type=out_t, mesh=_mesh(),
                     scratch_types=scratch, name=f"seg_rows_{feat}_{cpt}")


@functools.lru_cache(maxsize=None)
def _make_he_seg_sum(n_nodes, cpt, rows_pad):
    ro_per_tile, _ = _ro_split(n_nodes)
    z_per_tile = rows_pad // NS
    assert z_per_tile % LANES == 0 and ro_per_tile <= z_per_tile

    scratch = [
        pltpu.VMEM((cpt, CH), jnp.float32),
        pltpu.VMEM((cpt, CH), jnp.int32),
        pltpu.VMEM((z_per_tile,), jnp.float32),
        pltpu.VMEM_SHARED((rows_pad,), jnp.float32),
    ]

    def body(val_hbm, dst_hbm, out_hbm, val_v, dst_v, z_v, acc):
        c = lax.axis_index("c")
        s = lax.axis_index("s")
        w = c * NS + s

        def zi(i, carry):
            z_v[pl.ds(i * LANES, LANES)] = jnp.zeros((LANES,), jnp.float32)
            return carry
        lax.fori_loop(0, z_per_tile // LANES, zi, 0)
        pltpu.sync_copy(z_v, acc.at[pl.ds(s * z_per_tile, z_per_tile)])

        base = w * cpt
        pltpu.sync_copy(dst_hbm.at[pl.ds(base, cpt)], dst_v)
        pltpu.sync_copy(val_hbm.at[pl.ds(base, cpt)], val_v)
        plsc.subcore_barrier()

        def chunk(j, carry):
            pltpu.sync_copy(val_v.at[j], acc.at[dst_v.at[j]], add=True)
            return carry
        lax.fori_loop(0, cpt, chunk, 0)
        plsc.subcore_barrier()

        pltpu.sync_copy(acc.at[pl.ds(s * ro_per_tile, ro_per_tile)],
                        z_v.at[pl.ds(0, ro_per_tile)])
        pltpu.sync_copy(z_v.at[pl.ds(0, ro_per_tile)],
                        out_hbm.at[pl.ds((c * NS + s) * ro_per_tile,
                                         ro_per_tile)])

    out_t = jax.ShapeDtypeStruct((NC * NS * ro_per_tile,), jnp.float32)
    return pl.kernel(body, out_type=out_t, mesh=_mesh(),
                     scratch_types=scratch, name=f"seg_elem_{cpt}")


def _self_body(hn_ref, he_ref, ws_ref, we_ref, b_ref, out_ref):
    acc = jnp.dot(hn_ref[...], ws_ref[...], preferred_element_type=jnp.float32,
                  precision=lax.Precision.HIGHEST)
    out_ref[...] = acc + he_ref[...] * we_ref[...] + b_ref[...]


def _combine_body(act, self_ref, agg_ref, wn_ref, out_ref):
    agg = agg_ref[0] + agg_ref[1]
    acc = self_ref[...] + jnp.dot(agg, wn_ref[...],
                                  preferred_element_type=jnp.float32,
                                  precision=lax.Precision.HIGHEST)
    if act == "relu":
        acc = jnp.maximum(acc, 0.0)
    elif act == "sigmoid":
        acc = jax.nn.sigmoid(acc)
    out_ref[...] = acc


def _apply_self(hn, he128, W, b, bm):
    n, d = hn.shape
    dout = W.shape[0]
    WsT = W[:, :d].T
    we = W[:, 2 * d][None, :]
    b2 = b[None, :]
    return pl.pallas_call(
        _self_body,
        grid=(n // bm,),
        in_specs=[
            pl.BlockSpec((bm, d), lambda i: (i, 0)),
            pl.BlockSpec((bm, d), lambda i: (i, 0)),
            pl.BlockSpec((d, dout), lambda i: (0, 0)),
            pl.BlockSpec((1, dout), lambda i: (0, 0)),
            pl.BlockSpec((1, dout), lambda i: (0, 0)),
        ],
        out_specs=pl.BlockSpec((bm, dout), lambda i: (i, 0)),
        out_shape=jax.ShapeDtypeStruct((n, dout), jnp.float32),
    )(hn, he128, WsT, we, b2)


def _apply_combine(selfT, aggP, W, b, act, bm):
    n, d = selfT.shape
    dout = W.shape[0]
    WnT = W[:, d:2 * d].T
    return pl.pallas_call(
        functools.partial(_combine_body, act),
        grid=(n // bm,),
        in_specs=[
            pl.BlockSpec((bm, dout), lambda i: (i, 0)),
            pl.BlockSpec((2, bm, d), lambda i: (0, i, 0)),
            pl.BlockSpec((d, dout), lambda i: (0, 0)),
        ],
        out_specs=pl.BlockSpec((bm, dout), lambda i: (i, 0)),
        out_shape=jax.ShapeDtypeStruct((n, dout), jnp.float32),
    )(selfT, aggP, WnT)


def kernel(node_feat, edge_index, edge_feat, W1, b1, W2, b2, W3, b3, W4, b4,
           W5, b5, W6, b6, W7, b7):
    n, d = node_feat.shape
    e = edge_index.shape[1]
    cpt = _ceil_to(-(-e // (CH * NW)), 8)
    n_chunks = cpt * NW
    e_pad = n_chunks * CH
    pad = e_pad - e

    ro_per_tile, _ = _ro_split(n)
    rows_pad = _ceil_to(max(n + 1, NS * ro_per_tile), NS * 128)

    src = edge_index[0]
    dst = edge_index[1]
    pad_idx = jnp.arange(pad, dtype=jnp.int32)
    srcp = jnp.concatenate([src, pad_idx % n]).reshape(n_chunks, CH)
    dstp = jnp.concatenate([dst, n + pad_idx % (rows_pad - n)]
                           ).reshape(n_chunks, CH)
    hev2d = jnp.pad(edge_feat[:, 0].astype(jnp.float32),
                    (0, pad)).reshape(n_chunks, CH)

    seg_hn = _make_edge_seg_sum(n, d, cpt, rows_pad)
    seg_he = _make_he_seg_sum(n, cpt, rows_pad)

    he_flat = seg_he(hev2d, dstp)
    heP = he_flat.reshape(NC, NS * ro_per_tile)
    he_sum = heP[0, :n] + heP[1, :n]
    he128 = jnp.broadcast_to(he_sum[:, None], (n, d))

    bm = next(c for c in (512, 400, 256, 200, 128, 80, 40, 16, 8)
              if n % c == 0)

    layers = [(W1, b1, "relu"), (W2, b2, "relu"), (W3, b3, "relu"),
              (W4, b4, "relu"), (W5, b5, "relu"), (W6, b6, "sigmoid"),
              (W7, b7, "none")]
    h = node_feat
    for W, b, act in layers:
        aggP = seg_hn(srcp, dstp, h)
        selfT = _apply_self(h, he128, W, b, bm)
        h = _apply_combine(selfT, aggP, W, b, act, bm)
    return h
